# baseline (speedup 1.0000x reference)
import jax
import jax.numpy as jnp
from jax import lax
from jax.experimental import pallas as pl
from jax.experimental.pallas import tpu as pltpu
from jax.experimental.pallas import tpu_sc as plsc

VOCAB = 1000000
EMBED = 64
WIDE = 128
BATCH = 16384
CTX = 50

NC = 2
NS = 16
NW = NC * NS

CBLK = 128
NFULL = VOCAB // CBLK
NFULL_PER_W = NFULL // NW
TAIL_COLS = VOCAB - NFULL * CBLK

ROWS_PER_DMA = 4
CHUNK = ROWS_PER_DMA * CTX
RPW = BATCH // NW
CPW = RPW // ROWS_PER_DMA
NGRP = CPW
NLANE = EMBED // 16
INV_CTX = 1.0 / CTX


def _transpose_body(tt_hbm, tail_hbm, t2_hbm, in_v, out_v, isem0, isem1, osem0, osem1):
    wid = lax.axis_index("s") * NC + lax.axis_index("c")
    iota = lax.broadcasted_iota(jnp.int32, (16,), 0)
    iota128 = iota * CBLK

    isems = (isem0, isem1)
    osems = (osem0, osem1)

    def blk(i):
        return i * NW + wid

    def issue_in(i, parity):
        pltpu.make_async_copy(
            tt_hbm.at[:, pl.ds(blk(i) * CBLK, CBLK)],
            in_v.at[parity],
            isems[parity],
        ).start()

    def wait_in(i, parity):
        pltpu.make_async_copy(
            tt_hbm.at[:, pl.ds(blk(i) * CBLK, CBLK)],
            in_v.at[parity],
            isems[parity],
        ).wait()

    def issue_out(i, parity):
        pltpu.make_async_copy(
            out_v.at[parity],
            t2_hbm.at[pl.ds(blk(i) * EMBED, EMBED)],
            osems[parity],
        ).start()

    def wait_out(i, parity):
        pltpu.make_async_copy(
            out_v.at[parity],
            t2_hbm.at[pl.ds(blk(i) * EMBED, EMBED)],
            osems[parity],
        ).wait()

    def transpose_block(parity):
        inb = in_v.at[parity]

        def col_body(r, carry):
            for cg in range(EMBED // 16):
                v = plsc.load_gather(inb, [iota + 16 * cg, jnp.full((16,), r, jnp.int32)])
                out_v[parity, lax.div(r, 2), pl.ds(lax.rem(r, 2) * EMBED + 16 * cg, 16)] = v
            return carry

        lax.fori_loop(0, CBLK, col_body, 0, unroll=False)

    issue_in(0, 0)

    def loop_body(i, carry):
        parity = lax.rem(i, 2)

        @pl.when(i + 1 < NFULL_PER_W)
        def _issue_next():
            nparity = lax.rem(i + 1, 2)

            @pl.when(nparity == 0)
            def _():
                issue_in(i + 1, 0)

            @pl.when(nparity == 1)
            def _():
                issue_in(i + 1, 1)

        @pl.when(parity == 0)
        def _p0():
            wait_in(i, 0)

            @pl.when(i >= 2)
            def _():
                wait_out(i - 2, 0)
            transpose_block(0)
            issue_out(i, 0)

        @pl.when(parity == 1)
        def _p1():
            wait_in(i, 1)

            @pl.when(i >= 2)
            def _():
                wait_out(i - 2, 1)
            transpose_block(1)
            issue_out(i, 1)

        return carry

    lax.fori_loop(0, NFULL_PER_W, loop_body, 0, unroll=False)
    wait_out(NFULL_PER_W - 2, (NFULL_PER_W - 2) % 2)
    wait_out(NFULL_PER_W - 1, (NFULL_PER_W - 1) % 2)

    @pl.when(wid < NFULL - NW * NFULL_PER_W)
    def _extra():
        b = NFULL_PER_W * NW + wid
        pltpu.sync_copy(tt_hbm.at[:, pl.ds(b * CBLK, CBLK)], in_v.at[0])
        transpose_block(0)
        pltpu.sync_copy(out_v.at[0], t2_hbm.at[pl.ds(b * EMBED, EMBED)])

    @pl.when(wid == 0)
    def _tail():
        pltpu.sync_copy(tail_hbm, out_v.at[0, pl.ds(0, TAIL_COLS // 2)])
        pltpu.sync_copy(out_v.at[0, pl.ds(0, TAIL_COLS // 2)],
                        t2_hbm.at[pl.ds(NFULL * EMBED, TAIL_COLS // 2)])


def _cbow_body(pidx_hbm, par_hbm, table_hbm, out_hbm,
               pidx_v, par_v, buf_v, out_v, sem0, sem1):
    wid = lax.axis_index("s") * NC + lax.axis_index("c")

    pltpu.sync_copy(pidx_hbm.at[pl.ds(wid * CPW * CHUNK, CPW * CHUNK)], pidx_v)
    pltpu.sync_copy(par_hbm.at[pl.ds(wid * RPW * 2, RPW * 2)], par_v)

    sems = (sem0, sem1)
    iota = lax.broadcasted_iota(jnp.int32, (16,), 0)
    base_q = [iota + 16 * q for q in range(NLANE)]

    def issue(c, parity):
        pltpu.make_async_copy(
            table_hbm.at[pidx_v.at[pl.ds(c * CHUNK, CHUNK)]],
            buf_v.at[parity, 0],
            sems[parity],
        ).start()

    def drain(c, parity):
        pltpu.make_async_copy(
            table_hbm.at[pidx_v.at[pl.ds(c * CHUNK, CHUNK)]],
            buf_v.at[parity, 0],
            sems[parity],
        ).wait()

    def reduce_chunk(g, parity):
        buf = buf_v.at[parity, 0]

        def row_body(rr, carry):
            orow = g * ROWS_PER_DMA + rr
            w0 = plsc.load_gather(par_v, [jnp.full((16,), 2 * orow, jnp.int32)])
            w1 = plsc.load_gather(par_v, [jnp.full((16,), 2 * orow + 1, jnp.int32)])
            acc = [None] * NLANE
            for j in range(CTX):
                w, sh = (w0, j) if j < 32 else (w1, j - 32)
                poff = lax.shift_left(
                    lax.bitwise_and(lax.shift_right_logical(w, sh), 1), 6)
                rvec = jnp.full((16,), rr * CTX + j, jnp.int32)
                for q in range(NLANE):
                    g_q = plsc.load_gather(buf, [rvec, poff + base_q[q]])
                    acc[q] = g_q if acc[q] is None else acc[q] + g_q
            for q in range(NLANE):
                out_v[pl.ds(orow * EMBED + 16 * q, 16)] = acc[q] * INV_CTX
            return carry

        lax.fori_loop(0, ROWS_PER_DMA, row_body, 0, unroll=False)

    issue(0, 0)

    def group_body(g, carry):
        parity = lax.rem(g, 2)

        @pl.when(g + 1 < NGRP)
        def _issue_next():
            nparity = lax.rem(g + 1, 2)

            @pl.when(nparity == 0)
            def _():
                issue(g + 1, 0)

            @pl.when(nparity == 1)
            def _():
                issue(g + 1, 1)

        @pl.when(parity == 0)
        def _p0():
            drain(g, 0)
            reduce_chunk(g, 0)

        @pl.when(parity == 1)
        def _p1():
            drain(g, 1)
            reduce_chunk(g, 1)

        return carry

    lax.fori_loop(0, NGRP, group_body, 0, unroll=False)

    pltpu.sync_copy(out_v, out_hbm.at[pl.ds(wid * RPW * EMBED, RPW * EMBED)])


@jax.jit
def _cbow(pidx, parw, tableT, tail2):
    mesh = plsc.VectorSubcoreMesh(core_axis_name="c", subcore_axis_name="s")
    params = pltpu.CompilerParams(
        use_tc_tiling_on_sc=True, needs_layout_passes=False)

    t2 = pl.kernel(
        _transpose_body,
        out_type=jax.ShapeDtypeStruct((VOCAB // 2, WIDE), jnp.float32),
        mesh=mesh,
        scratch_types=[
            pltpu.VMEM((2, EMBED, CBLK), jnp.float32),
            pltpu.VMEM((2, EMBED, WIDE), jnp.float32),
            pltpu.SemaphoreType.DMA,
            pltpu.SemaphoreType.DMA,
            pltpu.SemaphoreType.DMA,
            pltpu.SemaphoreType.DMA,
        ],
        compiler_params=params,
    )(tableT, tail2)

    out = pl.kernel(
        _cbow_body,
        out_type=jax.ShapeDtypeStruct((BATCH * EMBED,), jnp.float32),
        mesh=mesh,
        scratch_types=[
            pltpu.VMEM((CPW * CHUNK,), jnp.int32),
            pltpu.VMEM((RPW * 2,), jnp.int32),
            pltpu.VMEM((2, 1, CHUNK, WIDE), jnp.float32),
            pltpu.VMEM((RPW * EMBED,), jnp.float32),
            pltpu.SemaphoreType.DMA,
            pltpu.SemaphoreType.DMA,
        ],
        compiler_params=params,
    )(pidx, parw, t2)
    return out


def kernel(inputs, table):
    idx = inputs.astype(jnp.int32)
    pidx = lax.shift_right_logical(idx, 1).reshape(-1)
    par = lax.bitwise_and(idx, 1)
    sh = jnp.arange(32, dtype=jnp.int32)
    w0 = lax.shift_left(par[:, :32], sh[None, :]).sum(axis=1)
    w1 = lax.shift_left(par[:, 32:], sh[None, :CTX - 32]).sum(axis=1)
    parw = jnp.stack([w0, w1], axis=1).reshape(-1)
    tail2 = table[VOCAB - TAIL_COLS:].reshape(TAIL_COLS // 2, WIDE)
    return _cbow(pidx, parw, table.T, tail2).reshape(BATCH, EMBED)
